# Initial kernel scaffold; baseline (speedup 1.0000x reference)
import jax
import jax.numpy as jnp
from jax.experimental import pallas as pl


def kernel(pred, target):
    raise NotImplementedError("write your pallas kernel here")


# speedup vs baseline: 3.1144x; 3.1144x over previous
import functools

import jax
import jax.numpy as jnp
from jax import lax
from jax.experimental import pallas as pl
from jax.experimental.pallas import tpu as pltpu
from jax.experimental.pallas import tpu_sc as plsc

N, C, S = 2048, 8, 2048
L = 16
NCORES = 2
NSUB = 16
CLS_PER_CORE = C // NCORES
ROWS_PER_TILE = N // NSUB
CHUNKS = S // L

EPS = 1.1920928955078125e-07
SCALE = float(100.0 / (N * S + EPS))

_i32 = jnp.int32


def _argmax_body(pred_hbm, lab_hbm, rowbuf, labbuf):
    core = lax.axis_index("c")
    sid = lax.axis_index("s")
    iota = lax.iota(jnp.int32, L)
    neg_inf = jnp.full((L,), -jnp.inf, dtype=jnp.float32)
    zeros_i = jnp.zeros((L,), dtype=jnp.int32)

    for cl in range(CLS_PER_CORE):
        c = core * _i32(CLS_PER_CORE) + _i32(cl)

        def grp_body(g, _, c=c):
            base_row = sid * _i32(ROWS_PER_TILE) + g * _i32(L)
            lab_vec = zeros_i
            for j in range(L):
                pltpu.sync_copy(pred_hbm.at[base_row + _i32(j), c], rowbuf)

                def chunk_body(k, carry):
                    maxv, maxi = carry
                    v = rowbuf[pl.ds(k * _i32(L), L)]
                    idx = iota + k * _i32(L)
                    m = v > maxv
                    return (jnp.where(m, v, maxv), jnp.where(m, idx, maxi))

                maxv, maxi = lax.fori_loop(_i32(0), _i32(CHUNKS),
                                           chunk_body, (neg_inf, zeros_i))
                mval = jnp.max(maxv)
                cand = jnp.where(maxv == mval, maxi, _i32(S))
                lab_vec = jnp.where(iota == _i32(j), jnp.min(cand), lab_vec)
            labbuf[pl.ds(g * _i32(L), L)] = lab_vec
            return _i32(0)

        lax.fori_loop(_i32(0), _i32(ROWS_PER_TILE // L), grp_body, _i32(0))
        pltpu.sync_copy(labbuf,
                        lab_hbm.at[c, pl.ds(sid * _i32(ROWS_PER_TILE),
                                            ROWS_PER_TILE)])


def _count_body(target_hbm, lab_hbm, cnt_hbm, tbuf, labtile, cntbuf):
    core = lax.axis_index("c")
    sid = lax.axis_index("s")
    iota = lax.iota(jnp.int32, L)
    zeros_i = jnp.zeros((L,), dtype=jnp.int32)

    cnt_vec = zeros_i
    for cl in range(CLS_PER_CORE):
        c = core * _i32(CLS_PER_CORE) + _i32(cl)
        pltpu.sync_copy(lab_hbm.at[c], labtile)

        def n_body(r, acc, c=c):
            n = sid * _i32(ROWS_PER_TILE) + r
            pltpu.sync_copy(target_hbm.at[n, c], tbuf)

            def chunk_body(k, acc):
                t = tbuf[pl.ds(k * _i32(L), L)]
                lab = labtile[pl.ds(k * _i32(L), L)]
                return acc + (t == lab).astype(jnp.int32)

            return lax.fori_loop(_i32(0), _i32(CHUNKS), chunk_body, acc)

        acc = lax.fori_loop(_i32(0), _i32(ROWS_PER_TILE), n_body, zeros_i)
        cnt_vec = jnp.where(iota == _i32(cl),
                            jnp.sum(acc, dtype=jnp.int32), cnt_vec)

    cntbuf[...] = cnt_vec
    pltpu.sync_copy(cntbuf, cnt_hbm.at[core, sid])


def _combine_body(cnt_hbm, out_hbm, parttile, outbuf):
    core = lax.axis_index("c")
    sid = lax.axis_index("s")

    @pl.when(sid == 0)
    def _():
        pltpu.sync_copy(cnt_hbm.at[core], parttile)
        total = jnp.zeros((L,), dtype=jnp.int32)
        for i in range(NSUB):
            total = total + parttile[_i32(i)]
        vals = (total.astype(jnp.float32) + EPS) * SCALE
        outbuf[...] = vals
        pltpu.sync_copy(outbuf, out_hbm.at[core])


def _mesh():
    return plsc.VectorSubcoreMesh(core_axis_name="c", subcore_axis_name="s")


@jax.jit
def _sc_accuracy(pred, target):
    params = pltpu.CompilerParams(needs_layout_passes=False)
    labels = functools.partial(
        pl.kernel,
        out_type=jax.ShapeDtypeStruct((C, S), jnp.int32),
        mesh=_mesh(),
        compiler_params=params,
        scratch_types=[
            pltpu.VMEM((S,), jnp.float32),
            pltpu.VMEM((ROWS_PER_TILE,), jnp.int32),
        ],
    )(_argmax_body)(pred)

    counts = functools.partial(
        pl.kernel,
        out_type=jax.ShapeDtypeStruct((NCORES, NSUB, L), jnp.int32),
        mesh=_mesh(),
        compiler_params=params,
        scratch_types=[
            pltpu.VMEM((S,), jnp.int32),
            pltpu.VMEM((S,), jnp.int32),
            pltpu.VMEM((L,), jnp.int32),
        ],
    )(_count_body)(target, labels)

    return functools.partial(
        pl.kernel,
        out_type=jax.ShapeDtypeStruct((NCORES, L), jnp.float32),
        mesh=_mesh(),
        compiler_params=params,
        scratch_types=[
            pltpu.VMEM((NSUB, L), jnp.int32),
            pltpu.VMEM((L,), jnp.float32),
        ],
    )(_combine_body)(counts)


def kernel(pred, target):
    target = target.astype(jnp.int32)
    raw = _sc_accuracy(pred, target)
    return jnp.concatenate([raw[0, :CLS_PER_CORE],
                            raw[1, :CLS_PER_CORE]]).reshape(C, 1)


# speedup vs baseline: 4.1922x; 1.3461x over previous
import functools

import jax
import jax.numpy as jnp
from jax import lax
from jax.experimental import pallas as pl
from jax.experimental.pallas import tpu as pltpu
from jax.experimental.pallas import tpu_sc as plsc

N, C, S = 2048, 8, 2048
L = 16
NCORES = 2
NSUB = 16
CLS_PER_CORE = C // NCORES
SPAN = S // NSUB
CHUNKS = S // L
PBLK = 8
NBLK = 256
KCH = SPAN // L

EPS = 1.1920928955078125e-07
SCALE = float(100.0 / (N * S + EPS))

_i32 = jnp.int32


def _row_argmax(buf, r, iota, neg_inf, zeros_i):
    def chunk_body(_, carry):
        maxv, maxk, k = carry
        v = buf[_i32(r), pl.ds(k * _i32(L), L)]
        m = v > maxv
        return (jnp.where(m, v, maxv), jnp.where(m, k, maxk), k + _i32(1))

    maxv, maxk, _ = lax.fori_loop(0, CHUNKS, chunk_body,
                                  (neg_inf, zeros_i, _i32(0)), unroll=8)
    mval = jnp.max(maxv)
    cand = jnp.where(maxv == mval, maxk * _i32(L) + iota, _i32(S))
    return jnp.min(cand)


def _count_block(buf, labk, acc):
    def row_body(_, carry):
        a0, a1, r = carry
        for k in range(KCH):
            t = buf[r, pl.ds(_i32(k * L), L)]
            eq = (t == labk[k]).astype(jnp.int32)
            if k % 2 == 0:
                a0 = a0 + eq
            else:
                a1 = a1 + eq
        return (a0, a1, r + _i32(1))

    a0, a1, _ = lax.fori_loop(0, NBLK, row_body, (*acc, _i32(0)), unroll=4)
    return (a0, a1)


def _main_body(pred_hbm, target_hbm, cnt_hbm,
               pbuf0, pbuf1, tbuf0, tbuf1, labbuf, cntbuf,
               psem0, psem1, tsem0, tsem1):
    core = lax.axis_index("c")
    sid = lax.axis_index("s")
    iota = lax.iota(jnp.int32, L)
    neg_inf = jnp.full((L,), -jnp.inf, dtype=jnp.float32)
    zeros_i = jnp.zeros((L,), dtype=jnp.int32)
    s0 = sid * _i32(SPAN)

    cnt_vec = zeros_i
    for cl in range(CLS_PER_CORE):
        c = core * _i32(CLS_PER_CORE) + _i32(cl)

        def pstart(blk, buf, sem):
            base = jnp.minimum(s0 + blk * _i32(PBLK), _i32(N - PBLK))
            return pltpu.async_copy(
                pred_hbm.at[pl.ds(base, PBLK), c], buf, sem)

        def pwait(sem):
            pltpu.make_async_copy(
                pred_hbm.at[pl.ds(_i32(0), PBLK), _i32(0)], pbuf0, sem).wait()

        pstart(_i32(0), pbuf0, psem0)

        def pgrp(g, _, c=c):
            pstart(_i32(2) * g + _i32(1), pbuf1, psem1)
            pwait(psem0)
            lab_vec = zeros_i
            for r in range(PBLK):
                lab = _row_argmax(pbuf0, r, iota, neg_inf, zeros_i)
                lab_vec = jnp.where(iota == _i32(r), lab, lab_vec)
            pstart(_i32(2) * g + _i32(2), pbuf0, psem0)
            pwait(psem1)
            for r in range(PBLK):
                lab = _row_argmax(pbuf1, r, iota, neg_inf, zeros_i)
                lab_vec = jnp.where(iota == _i32(PBLK + r), lab, lab_vec)
            labbuf[pl.ds(g * _i32(L), L)] = lab_vec
            return _i32(0)

        lax.fori_loop(_i32(0), _i32(SPAN // (2 * PBLK)), pgrp, _i32(0))
        pwait(psem0)

        labk = [labbuf[pl.ds(_i32(k * L), L)] for k in range(KCH)]

        def tstart(blk, buf, sem):
            base = jnp.minimum(blk, _i32(N // NBLK - 1)) * _i32(NBLK)
            return pltpu.async_copy(
                target_hbm.at[pl.ds(base, NBLK), c, pl.ds(s0, SPAN)],
                buf, sem)

        def twait(sem):
            pltpu.make_async_copy(
                target_hbm.at[pl.ds(_i32(0), NBLK), _i32(0),
                              pl.ds(_i32(0), SPAN)], tbuf0, sem).wait()

        tstart(_i32(0), tbuf0, tsem0)

        def tgrp(g, acc, c=c, labk=labk):
            tstart(_i32(2) * g + _i32(1), tbuf1, tsem1)
            twait(tsem0)
            acc = _count_block(tbuf0, labk, acc)
            tstart(_i32(2) * g + _i32(2), tbuf0, tsem0)
            twait(tsem1)
            acc = _count_block(tbuf1, labk, acc)
            return acc

        a0, a1 = lax.fori_loop(_i32(0), _i32(N // NBLK // 2), tgrp,
                               (zeros_i, zeros_i))
        twait(tsem0)
        cnt_vec = jnp.where(iota == _i32(cl),
                            jnp.sum(a0 + a1, dtype=jnp.int32), cnt_vec)

    cntbuf[...] = cnt_vec
    pltpu.sync_copy(cntbuf, cnt_hbm.at[core, sid])


def _combine_tc(cnt_ref, out_ref):
    cnts = cnt_ref[...].astype(jnp.float32)
    tot = jnp.sum(cnts, axis=1)
    vals = (tot[:, :CLS_PER_CORE] + EPS) * SCALE
    out_ref[...] = vals


@jax.jit
def _sc_accuracy(pred, target):
    counts = functools.partial(
        pl.kernel,
        out_type=jax.ShapeDtypeStruct((NCORES, NSUB, L), jnp.int32),
        mesh=plsc.VectorSubcoreMesh(core_axis_name="c",
                                    subcore_axis_name="s"),
        compiler_params=pltpu.CompilerParams(needs_layout_passes=False),
        scratch_types=[
            pltpu.VMEM((PBLK, S), jnp.float32),
            pltpu.VMEM((PBLK, S), jnp.float32),
            pltpu.VMEM((NBLK, SPAN), jnp.int32),
            pltpu.VMEM((NBLK, SPAN), jnp.int32),
            pltpu.VMEM((SPAN,), jnp.int32),
            pltpu.VMEM((L,), jnp.int32),
            pltpu.SemaphoreType.DMA,
            pltpu.SemaphoreType.DMA,
            pltpu.SemaphoreType.DMA,
            pltpu.SemaphoreType.DMA,
        ],
    )(_main_body)(pred, target)

    return pl.pallas_call(
        _combine_tc,
        out_shape=jax.ShapeDtypeStruct((NCORES, CLS_PER_CORE), jnp.float32),
    )(counts)


def kernel(pred, target):
    target = target.astype(jnp.int32)
    return _sc_accuracy(pred, target).reshape(C, 1)


# speedup vs baseline: 4.4630x; 1.0646x over previous
import functools

import jax
import jax.numpy as jnp
from jax import lax
from jax.experimental import pallas as pl
from jax.experimental.pallas import tpu as pltpu
from jax.experimental.pallas import tpu_sc as plsc

N, C, S = 2048, 8, 2048
L = 16
NCORES = 2
NSUB = 16
CLS_PER_CORE = C // NCORES
SPAN = S // NSUB
CHUNKS = S // L
PBLK = 8

EPS = 1.1920928955078125e-07
SCALE = float(100.0 / (N * S + EPS))

_i32 = jnp.int32


def _row_argmax(buf, r, iota, neg_inf, zeros_i):
    def chunk_body(_, carry):
        maxv, maxk, k = carry
        v = buf[_i32(r), pl.ds(k * _i32(L), L)]
        m = v > maxv
        return (jnp.where(m, v, maxv), jnp.where(m, k, maxk), k + _i32(1))

    maxv, maxk, _ = lax.fori_loop(0, CHUNKS, chunk_body,
                                  (neg_inf, zeros_i, _i32(0)), unroll=8)
    mval = jnp.max(maxv)
    cand = jnp.where(maxv == mval, maxk * _i32(L) + iota, _i32(S))
    return jnp.min(cand)


def _argmax_body(pred_hbm, lab_hbm, pbuf0, pbuf1, labbuf, psem0, psem1):
    core = lax.axis_index("c")
    sid = lax.axis_index("s")
    iota = lax.iota(jnp.int32, L)
    neg_inf = jnp.full((L,), -jnp.inf, dtype=jnp.float32)
    zeros_i = jnp.zeros((L,), dtype=jnp.int32)
    s0 = sid * _i32(SPAN)

    for cl in range(CLS_PER_CORE):
        c = core * _i32(CLS_PER_CORE) + _i32(cl)

        def pstart(blk, buf, sem, c=c):
            base = jnp.minimum(s0 + blk * _i32(PBLK), _i32(N - PBLK))
            pltpu.async_copy(pred_hbm.at[pl.ds(base, PBLK), c], buf, sem)

        def pwait(sem):
            pltpu.make_async_copy(
                pred_hbm.at[pl.ds(_i32(0), PBLK), _i32(0)], pbuf0, sem).wait()

        pstart(_i32(0), pbuf0, psem0)

        def pgrp(g, _, c=c):
            pstart(_i32(2) * g + _i32(1), pbuf1, psem1, c=c)
            pwait(psem0)
            lab_vec = zeros_i
            for r in range(PBLK):
                lab = _row_argmax(pbuf0, r, iota, neg_inf, zeros_i)
                lab_vec = jnp.where(iota == _i32(r), lab, lab_vec)
            pstart(_i32(2) * g + _i32(2), pbuf0, psem0, c=c)
            pwait(psem1)
            for r in range(PBLK):
                lab = _row_argmax(pbuf1, r, iota, neg_inf, zeros_i)
                lab_vec = jnp.where(iota == _i32(PBLK + r), lab, lab_vec)
            labbuf[pl.ds(g * _i32(L), L)] = lab_vec
            return _i32(0)

        lax.fori_loop(_i32(0), _i32(SPAN // (2 * PBLK)), pgrp, _i32(0))
        pwait(psem0)
        pltpu.sync_copy(labbuf, lab_hbm.at[c, pl.ds(s0, SPAN)])


def _count_block(buf, labtile, acc):
    def chunk_body(_, carry):
        a0, a1, k = carry
        off = k * _i32(L)
        lab = labtile[pl.ds(off, L)]
        for r in range(PBLK):
            eq = (buf[_i32(r), pl.ds(off, L)] == lab).astype(jnp.int32)
            if r % 2 == 0:
                a0 = a0 + eq
            else:
                a1 = a1 + eq
        return (a0, a1, k + _i32(1))

    a0, a1, _ = lax.fori_loop(0, CHUNKS, chunk_body, (*acc, _i32(0)),
                              unroll=2)
    return (a0, a1)


def _count_body(target_hbm, lab_hbm, cnt_hbm,
                tbuf0, tbuf1, labtile, cntbuf, tsem0, tsem1):
    core = lax.axis_index("c")
    sid = lax.axis_index("s")
    iota = lax.iota(jnp.int32, L)
    zeros_i = jnp.zeros((L,), dtype=jnp.int32)
    n0 = sid * _i32(SPAN)

    cnt_vec = zeros_i
    for cl in range(CLS_PER_CORE):
        c = core * _i32(CLS_PER_CORE) + _i32(cl)
        pltpu.sync_copy(lab_hbm.at[c], labtile)

        def tstart(blk, buf, sem, c=c):
            base = jnp.minimum(n0 + blk * _i32(PBLK), _i32(N - PBLK))
            pltpu.async_copy(target_hbm.at[pl.ds(base, PBLK), c], buf, sem)

        def twait(sem):
            pltpu.make_async_copy(
                target_hbm.at[pl.ds(_i32(0), PBLK), _i32(0)],
                tbuf0, sem).wait()

        tstart(_i32(0), tbuf0, tsem0)

        def tgrp(g, acc, c=c):
            tstart(_i32(2) * g + _i32(1), tbuf1, tsem1, c=c)
            twait(tsem0)
            acc = _count_block(tbuf0, labtile, acc)
            tstart(_i32(2) * g + _i32(2), tbuf0, tsem0, c=c)
            twait(tsem1)
            acc = _count_block(tbuf1, labtile, acc)
            return acc

        a0, a1 = lax.fori_loop(_i32(0), _i32(SPAN // (2 * PBLK)), tgrp,
                               (zeros_i, zeros_i))
        twait(tsem0)
        cnt_vec = jnp.where(iota == _i32(cl),
                            jnp.sum(a0 + a1, dtype=jnp.int32), cnt_vec)

    cntbuf[...] = cnt_vec
    pltpu.sync_copy(cntbuf, cnt_hbm.at[core, sid])


def _combine_tc(cnt_ref, out_ref):
    cnts = cnt_ref[...].astype(jnp.float32)
    tot = jnp.sum(cnts, axis=1)
    vals = (tot[:, :CLS_PER_CORE] + EPS) * SCALE
    out_ref[...] = vals


def _mesh():
    return plsc.VectorSubcoreMesh(core_axis_name="c", subcore_axis_name="s")


@jax.jit
def _sc_accuracy(pred, target):
    params = pltpu.CompilerParams(needs_layout_passes=False)
    labels = functools.partial(
        pl.kernel,
        out_type=jax.ShapeDtypeStruct((C, S), jnp.int32),
        mesh=_mesh(),
        compiler_params=params,
        scratch_types=[
            pltpu.VMEM((PBLK, S), jnp.float32),
            pltpu.VMEM((PBLK, S), jnp.float32),
            pltpu.VMEM((SPAN,), jnp.int32),
            pltpu.SemaphoreType.DMA,
            pltpu.SemaphoreType.DMA,
        ],
    )(_argmax_body)(pred)

    counts = functools.partial(
        pl.kernel,
        out_type=jax.ShapeDtypeStruct((NCORES, NSUB, L), jnp.int32),
        mesh=_mesh(),
        compiler_params=params,
        scratch_types=[
            pltpu.VMEM((PBLK, S), jnp.int32),
            pltpu.VMEM((PBLK, S), jnp.int32),
            pltpu.VMEM((S,), jnp.int32),
            pltpu.VMEM((L,), jnp.int32),
            pltpu.SemaphoreType.DMA,
            pltpu.SemaphoreType.DMA,
        ],
    )(_count_body)(target, labels)

    return pl.pallas_call(
        _combine_tc,
        out_shape=jax.ShapeDtypeStruct((NCORES, CLS_PER_CORE), jnp.float32),
    )(counts)


def kernel(pred, target):
    target = target.astype(jnp.int32)
    return _sc_accuracy(pred, target).reshape(C, 1)
